# baseline (speedup 1.0000x reference)
import functools

import jax
import jax.numpy as jnp
from jax import lax
from jax.experimental import pallas as pl
from jax.experimental.pallas import tpu as pltpu
from jax.experimental.pallas import tpu_sc as plsc

N_EMBD = 768
BATCH = 4
SEQ = 1024
NC = 2
NS = 16
NW = NC * NS
CHUNK = SEQ // NW
HALF = CHUNK // 2
LANES = 16
COLS = N_EMBD // LANES

_mesh = plsc.VectorSubcoreMesh(core_axis_name="c", subcore_axis_name="s")


@functools.partial(
    pl.kernel,
    mesh=_mesh,
    out_type=jax.ShapeDtypeStruct((BATCH * SEQ, N_EMBD), jnp.float32),
    scratch_types=[
        pltpu.VMEM((BATCH, CHUNK), jnp.int32),
        pltpu.VMEM((BATCH, CHUNK, N_EMBD), jnp.float32),
        pltpu.VMEM((CHUNK, N_EMBD), jnp.float32),
        pltpu.SemaphoreType.DMA,
        pltpu.SemaphoreType.DMA,
        pltpu.SemaphoreType.DMA,
        pltpu.SemaphoreType.DMA,
    ],
)
def _embed(ids_hbm, tok_hbm, pos_hbm, out_hbm, idx_v, rows_v, pos_v,
           isem, psem, gsem, osem):
    wid = lax.axis_index("s") * NC + lax.axis_index("c")
    s_base = wid * CHUNK

    idx_cps = [
        pltpu.make_async_copy(
            ids_hbm.at[b, pl.ds(s_base, CHUNK)], idx_v.at[b], isem)
        for b in range(BATCH)
    ]
    for cp in idx_cps:
        cp.start()
    pos_cp = pltpu.make_async_copy(pos_hbm.at[pl.ds(s_base, CHUNK)], pos_v, psem)
    pos_cp.start()

    for cp in idx_cps:
        cp.wait()
    pieces = [(b, h) for b in range(BATCH) for h in range(2)]
    gathers = [
        pltpu.make_async_copy(
            tok_hbm.at[idx_v.at[b, pl.ds(h * HALF, HALF)]],
            rows_v.at[b, pl.ds(h * HALF, HALF)],
            gsem,
        )
        for b, h in pieces
    ]
    for cp in gathers:
        cp.start()
    pos_cp.wait()

    outs = [
        pltpu.make_async_copy(
            rows_v.at[b, pl.ds(h * HALF, HALF)],
            out_hbm.at[pl.ds(b * SEQ + s_base + h * HALF, HALF)],
            osem,
        )
        for b, h in pieces
    ]

    for p, (b, h) in enumerate(pieces):
        gathers[p].wait()

        @plsc.parallel_loop(h * HALF, (h + 1) * HALF, unroll=2)
        def _add_rows(r, b=b):
            for c in range(COLS):
                off = c * LANES
                rows_v[b, r, pl.ds(off, LANES)] = (
                    rows_v[b, r, pl.ds(off, LANES)] + pos_v[r, pl.ds(off, LANES)]
                )

        outs[p].start()
    for cp in outs:
        cp.wait()


def kernel(input_ids, tok_table, pos_table):
    out = _embed(input_ids.astype(jnp.int32), tok_table, pos_table)
    return out.reshape(BATCH, SEQ, N_EMBD)
